# Initial kernel scaffold; baseline (speedup 1.0000x reference)
import jax
import jax.numpy as jnp
from jax.experimental import pallas as pl


def kernel(inputs, targets, features):
    raise NotImplementedError("write your pallas kernel here")


# speedup vs baseline: 1.4911x; 1.4911x over previous
import jax
import jax.numpy as jnp
from jax.experimental import pallas as pl
from jax.experimental.pallas import tpu as pltpu

_NF = 64
_NS = 100000
_B = 1024
_TEMP = 0.05
_INV_TEMP = 1.0 / _TEMP
_BN = 2000


def _loss_body(x_ref, tgt_ref, f_ref, out_ref, s_acc, t_acc):
    i = pl.program_id(0)

    @pl.when(i == 0)
    def _init():
        s_acc[...] = jnp.zeros_like(s_acc)
        t_acc[...] = jnp.zeros_like(t_acc)

    x = x_ref[...]
    norm = jnp.sqrt(jnp.sum(x * x, axis=1, keepdims=True))
    xn = x / jnp.maximum(norm, 1e-12)

    logits = jax.lax.dot_general(
        xn, f_ref[...], (((1,), (1,)), ((), ()))) * _INV_TEMP
    s_acc[...] += jnp.sum(jnp.exp(logits - _INV_TEMP), axis=1, keepdims=True)

    cols = i * _BN + jax.lax.broadcasted_iota(jnp.int32, (_B, _BN), 1)
    hit = cols == tgt_ref[...]
    t_acc[...] += jnp.sum(jnp.where(hit, logits, 0.0), axis=1, keepdims=True)

    @pl.when(i == pl.num_programs(0) - 1)
    def _final():
        lse = jnp.log(s_acc[...]) + _INV_TEMP
        out_ref[...] = jnp.mean(lse - t_acc[...]).reshape(1, 1)


def kernel(inputs, targets, features):
    tgt2 = targets.reshape(_B, 1).astype(jnp.int32)
    out = pl.pallas_call(
        _loss_body,
        grid=(_NS // _BN,),
        in_specs=[
            pl.BlockSpec((_B, _NF), lambda i: (0, 0)),
            pl.BlockSpec((_B, 1), lambda i: (0, 0)),
            pl.BlockSpec((_BN, _NF), lambda i: (i, 0)),
        ],
        out_specs=pl.BlockSpec((1, 1), lambda i: (0, 0)),
        out_shape=jax.ShapeDtypeStruct((1, 1), jnp.float32),
        scratch_shapes=[
            pltpu.VMEM((_B, 1), jnp.float32),
            pltpu.VMEM((_B, 1), jnp.float32),
        ],
        compiler_params=pltpu.CompilerParams(
            dimension_semantics=("arbitrary",)),
    )(inputs, tgt2, features)
    return out[0, 0]


# speedup vs baseline: 1.7239x; 1.1561x over previous
import jax
import jax.numpy as jnp
from jax.experimental import pallas as pl
from jax.experimental.pallas import tpu as pltpu

_NF = 64
_NS = 100000
_B = 1024
_TEMP = 0.05
_INV_TEMP = 1.0 / _TEMP
_BN = 2000


def _loss_body(x_ref, tgt_ref, f_ref, out_ref, s_acc, t_acc):
    i = pl.program_id(0)

    @pl.when(i == 0)
    def _init():
        s_acc[...] = jnp.zeros_like(s_acc)
        t_acc[...] = jnp.zeros_like(t_acc)

    x = x_ref[...]
    norm = jnp.sqrt(jnp.sum(x * x, axis=1, keepdims=True))
    xn = x * (_INV_TEMP / jnp.maximum(norm, 1e-12))

    logits = jax.lax.dot_general(
        xn, f_ref[...], (((1,), (1,)), ((), ())))
    s_acc[...] += jnp.sum(jnp.exp(logits), axis=1, keepdims=True)

    cols = i * _BN + jax.lax.broadcasted_iota(jnp.int32, (_B, _BN), 1)
    hit = cols == tgt_ref[...]
    t_acc[...] += jnp.sum(jnp.where(hit, logits, 0.0), axis=1, keepdims=True)

    @pl.when(i == pl.num_programs(0) - 1)
    def _final():
        lse = jnp.log(s_acc[...])
        out_ref[...] = jnp.mean(lse - t_acc[...]).reshape(1, 1)


def kernel(inputs, targets, features):
    tgt2 = targets.reshape(_B, 1).astype(jnp.int32)
    out = pl.pallas_call(
        _loss_body,
        grid=(_NS // _BN,),
        in_specs=[
            pl.BlockSpec((_B, _NF), lambda i: (0, 0)),
            pl.BlockSpec((_B, 1), lambda i: (0, 0)),
            pl.BlockSpec((_BN, _NF), lambda i: (i, 0)),
        ],
        out_specs=pl.BlockSpec((1, 1), lambda i: (0, 0)),
        out_shape=jax.ShapeDtypeStruct((1, 1), jnp.float32),
        scratch_shapes=[
            pltpu.VMEM((_B, 1), jnp.float32),
            pltpu.VMEM((_B, 1), jnp.float32),
        ],
        compiler_params=pltpu.CompilerParams(
            dimension_semantics=("arbitrary",)),
    )(inputs, tgt2, features)
    return out[0, 0]
